# baseline (speedup 1.0000x reference)
import jax
import jax.numpy as jnp
from jax import lax
from jax.experimental import pallas as pl
from jax.experimental.pallas import tpu as pltpu
from jax.experimental.pallas import tpu_sc as plsc

H = 128
NGDF = 16
NSIG = 16
CHEM_DIM = 2 + NGDF * 2
GEOM_DIM = NGDF * 2 + NSIG
N_NODES = 50000
N_EDGES = 800000
EPS = 1e-5

E_BLK = 4000
N_BLK = 5000

SC_PARTS = 4
PART_BASE = (0, 12512, 25024, 37536, N_NODES)
ACC_ROWS = 12544
DUMMY_ROW = 12512
ZCHUNK = 784
T_EDGE = 112


def _sigmoid(x):
    return 1.0 / (1.0 + jnp.exp(-x))


def _silu(x):
    return x * _sigmoid(x)


def _softplus(x):
    return jnp.maximum(x, 0.0) + jnp.log(1.0 + jnp.exp(-jnp.abs(x)))


def _dot(a, b):
    return jnp.dot(a, b, preferred_element_type=jnp.float32)


def _dot_hi(a, b):
    return jnp.dot(a, b, preferred_element_type=jnp.float32,
                   precision=lax.Precision.HIGHEST)


def _acc_stats(acc_ref, h):
    acc_ref[0:1, :] += jnp.sum(h, axis=0, keepdims=True)
    acc_ref[1:2, :] += jnp.sum(h * h, axis=0, keepdims=True)


def _mk_ss(ss_ref, acc_ref, g_ref, be_ref, n):
    mean = acc_ref[0:1, :] / n
    var = acc_ref[1:2, :] / n - mean * mean
    s = g_ref[...] / jnp.sqrt(var + EPS)
    ss_ref[0:1, :] = s
    ss_ref[1:2, :] = be_ref[...] - mean * s


def _edge_mega_body(x_ref, w1_ref, b1_ref, g1_ref, be1_ref,
                    w2_ref, b2_ref, g2_ref, be2_ref, o_ref,
                    accx0, accx1, acc2, ss1, ss2):
    p = pl.program_id(0)
    j = pl.program_id(1)

    @pl.when((p == 0) & (j == 0))
    def _():
        accx0[...] = jnp.zeros_like(accx0)
        accx1[...] = jnp.zeros_like(accx1)

    @pl.when((p == 1) & (j == 0))
    def _():
        w1 = w1_ref[...]
        b1 = b1_ref[...]
        u = _dot_hi(accx0[...], w1) / N_EDGES
        q = jnp.sum(w1 * _dot_hi(accx1[...], w1), axis=0, keepdims=True)
        mean = u + b1
        var = q / N_EDGES + 2.0 * b1 * u + b1 * b1 - mean * mean
        s = g1_ref[...] / jnp.sqrt(var + EPS)
        ss1[0:1, :] = s
        ss1[1:2, :] = be1_ref[...] - mean * s
        acc2[...] = jnp.zeros_like(acc2)

    @pl.when((p == 2) & (j == 0))
    def _():
        _mk_ss(ss2, acc2, g2_ref, be2_ref, N_EDGES)

    @pl.when(p == 0)
    def _():
        xb = x_ref[...]
        accx0[...] += jnp.sum(xb, axis=0, keepdims=True)
        accx1[...] += lax.dot_general(
            xb, xb, (((0,), (0,)), ((), ())),
            preferred_element_type=jnp.float32,
            precision=lax.Precision.HIGHEST)

    @pl.when(p == 1)
    def _():
        h1 = _dot(x_ref[...], w1_ref[...]) + b1_ref[...]
        a1 = _silu(h1 * ss1[0:1, :] + ss1[1:2, :])
        h2 = _dot(a1, w2_ref[...]) + b2_ref[...]
        _acc_stats(acc2, h2)

    @pl.when(p == 2)
    def _():
        h1 = _dot(x_ref[...], w1_ref[...]) + b1_ref[...]
        a1 = _silu(h1 * ss1[0:1, :] + ss1[1:2, :])
        h2 = (_dot(a1, w2_ref[...]) + b2_ref[...]) * ss2[0:1, :] + ss2[1:2, :]
        o_ref[...] = _sigmoid(h2[:, :H]) * _softplus(h2[:, H:])


def _run_edge_mega(x, w1t, b1, g1, be1, w2t, b2, g2, be2):
    n = x.shape[0]
    nb = n // E_BLK
    return pl.pallas_call(
        _edge_mega_body,
        grid=(3, nb),
        in_specs=[
            pl.BlockSpec((E_BLK, CHEM_DIM), lambda p, j: (j, 0)),
            pl.BlockSpec((CHEM_DIM, H), lambda p, j: (0, 0)),
            _evec(H), _evec(H), _evec(H),
            pl.BlockSpec((H, 2 * H), lambda p, j: (0, 0)),
            _evec(2 * H), _evec(2 * H), _evec(2 * H),
        ],
        out_specs=pl.BlockSpec((E_BLK, H),
                               lambda p, j: (jnp.where(p == 2, j, 0), 0)),
        out_shape=jax.ShapeDtypeStruct((n, H), jnp.float32),
        scratch_shapes=[
            pltpu.VMEM((1, CHEM_DIM), jnp.float32),
            pltpu.VMEM((CHEM_DIM, CHEM_DIM), jnp.float32),
            pltpu.VMEM((2, 2 * H), jnp.float32),
            pltpu.VMEM((2, H), jnp.float32),
            pltpu.VMEM((2, 2 * H), jnp.float32),
        ],
    )(x, w1t, b1, g1, be1, w2t, b2, g2, be2)


def _evec(d):
    return pl.BlockSpec((1, d), lambda p, j: (0, 0))


def _node_mega_body(g_ref, agg_ref, gw1_ref, gb1_ref, gg1_ref, gbe1_ref,
                    gw2_ref, gb2_ref, gg2_ref, gbe2_ref,
                    fwa_ref, fwb_ref, fb1_ref, fg1_ref, fbe1_ref,
                    fw2_ref, fb2_ref, fg2_ref, fbe2_ref, o_ref,
                    ag1, ag2, af1, af2, sg1, sg2, sf1, sf2):
    p = pl.program_id(0)
    j = pl.program_id(1)

    @pl.when((p == 0) & (j == 0))
    def _():
        ag1[...] = jnp.zeros_like(ag1)

    @pl.when((p == 1) & (j == 0))
    def _():
        _mk_ss(sg1, ag1, gg1_ref, gbe1_ref, N_NODES)
        ag2[...] = jnp.zeros_like(ag2)

    @pl.when((p == 2) & (j == 0))
    def _():
        _mk_ss(sg2, ag2, gg2_ref, gbe2_ref, N_NODES)
        af1[...] = jnp.zeros_like(af1)

    @pl.when((p == 3) & (j == 0))
    def _():
        _mk_ss(sf1, af1, fg1_ref, fbe1_ref, N_NODES)
        af2[...] = jnp.zeros_like(af2)

    @pl.when((p == 4) & (j == 0))
    def _():
        _mk_ss(sf2, af2, fg2_ref, fbe2_ref, N_NODES)

    def hg1_raw():
        return _dot(g_ref[...], gw1_ref[...]) + gb1_ref[...]

    def hg2n():
        a = _silu(hg1_raw() * sg1[0:1, :] + sg1[1:2, :])
        h = _dot(a, gw2_ref[...]) + gb2_ref[...]
        return h * sg2[0:1, :] + sg2[1:2, :]

    def f1_raw():
        return (_dot(agg_ref[...], fwa_ref[...])
                + _dot(hg2n(), fwb_ref[...]) + fb1_ref[...])

    def f2_raw():
        a = _silu(f1_raw() * sf1[0:1, :] + sf1[1:2, :])
        return _dot(a, fw2_ref[...]) + fb2_ref[...]

    @pl.when(p == 0)
    def _():
        _acc_stats(ag1, hg1_raw())

    @pl.when(p == 1)
    def _():
        a = _silu(hg1_raw() * sg1[0:1, :] + sg1[1:2, :])
        _acc_stats(ag2, _dot(a, gw2_ref[...]) + gb2_ref[...])

    @pl.when(p == 2)
    def _():
        _acc_stats(af1, f1_raw())

    @pl.when(p == 3)
    def _():
        _acc_stats(af2, f2_raw())

    @pl.when(p == 4)
    def _():
        o_ref[...] = f2_raw() * sf2[0:1, :] + sf2[1:2, :]


def _run_node_mega(g, agg, gw1t, gb1, gg1, gbe1, gw2t, gb2, gg2, gbe2,
                   fwa, fwb, fb1, fg1, fbe1, fw2t, fb2, fg2, fbe2):
    nb = N_NODES // N_BLK
    h2 = H // 2
    return pl.pallas_call(
        _node_mega_body,
        grid=(5, nb),
        in_specs=[
            pl.BlockSpec((N_BLK, GEOM_DIM), lambda p, j: (j, 0)),
            pl.BlockSpec((N_BLK, H), lambda p, j: (j, 0)),
            pl.BlockSpec((GEOM_DIM, h2), lambda p, j: (0, 0)),
            _evec(h2), _evec(h2), _evec(h2),
            pl.BlockSpec((h2, h2), lambda p, j: (0, 0)),
            _evec(h2), _evec(h2), _evec(h2),
            pl.BlockSpec((H, H), lambda p, j: (0, 0)),
            pl.BlockSpec((h2, H), lambda p, j: (0, 0)),
            _evec(H), _evec(H), _evec(H),
            pl.BlockSpec((H, H), lambda p, j: (0, 0)),
            _evec(H), _evec(H), _evec(H),
        ],
        out_specs=pl.BlockSpec((N_BLK, H),
                               lambda p, j: (jnp.where(p == 4, j, 0), 0)),
        out_shape=jax.ShapeDtypeStruct((N_NODES, H), jnp.float32),
        scratch_shapes=[
            pltpu.VMEM((2, h2), jnp.float32),
            pltpu.VMEM((2, h2), jnp.float32),
            pltpu.VMEM((2, H), jnp.float32),
            pltpu.VMEM((2, H), jnp.float32),
            pltpu.VMEM((2, h2), jnp.float32),
            pltpu.VMEM((2, h2), jnp.float32),
            pltpu.VMEM((2, H), jnp.float32),
            pltpu.VMEM((2, H), jnp.float32),
        ],
    )(g, agg, gw1t, gb1, gg1, gbe1, gw2t, gb2, gg2, gbe2,
      fwa, fwb, fb1, fg1, fbe1, fw2t, fb2, fg2, fbe2)


def _stats_mm1_body(x_ref, w_ref, b_ref, o_ref, acc_ref):
    i = pl.program_id(0)

    @pl.when(i == 0)
    def _():
        acc_ref[...] = jnp.zeros_like(acc_ref)

    h = _dot(x_ref[...], w_ref[...]) + b_ref[...]
    _acc_stats(acc_ref, h)

    @pl.when(i == pl.num_programs(0) - 1)
    def _():
        o_ref[...] = acc_ref[...]


def _stats_mm2_body(x_ref, w1_ref, b1_ref, s1_ref, t1_ref, w2_ref, b2_ref,
                    o_ref, acc_ref):
    i = pl.program_id(0)

    @pl.when(i == 0)
    def _():
        acc_ref[...] = jnp.zeros_like(acc_ref)

    h1 = _dot(x_ref[...], w1_ref[...]) + b1_ref[...]
    a1 = _silu(h1 * s1_ref[...] + t1_ref[...])
    h2 = _dot(a1, w2_ref[...]) + b2_ref[...]
    _acc_stats(acc_ref, h2)

    @pl.when(i == pl.num_programs(0) - 1)
    def _():
        o_ref[...] = acc_ref[...]


def _echem_body(x_ref, w1_ref, b1_ref, s1_ref, t1_ref, w2_ref, b2_ref,
                s2_ref, t2_ref, o_ref):
    h1 = _dot(x_ref[...], w1_ref[...]) + b1_ref[...]
    a1 = _silu(h1 * s1_ref[...] + t1_ref[...])
    h2 = (_dot(a1, w2_ref[...]) + b2_ref[...]) * s2_ref[...] + t2_ref[...]
    o_ref[...] = _sigmoid(h2[:, :H]) * _softplus(h2[:, H:])


def _vec_spec(d):
    return pl.BlockSpec((1, d), lambda i: (0, 0))


def _run_stats_mm1(x, w_t, b, blk):
    n, din = x.shape
    dout = w_t.shape[1]
    return pl.pallas_call(
        _stats_mm1_body,
        grid=(n // blk,),
        in_specs=[
            pl.BlockSpec((blk, din), lambda i: (i, 0)),
            pl.BlockSpec((din, dout), lambda i: (0, 0)),
            _vec_spec(dout),
        ],
        out_specs=pl.BlockSpec((2, dout), lambda i: (0, 0)),
        out_shape=jax.ShapeDtypeStruct((2, dout), jnp.float32),
        scratch_shapes=[pltpu.VMEM((2, dout), jnp.float32)],
    )(x, w_t, b)


def _run_stats_mm2(x, w1_t, b1, s1, t1, w2_t, b2, blk):
    n, din = x.shape
    dmid = w1_t.shape[1]
    dout = w2_t.shape[1]
    return pl.pallas_call(
        _stats_mm2_body,
        grid=(n // blk,),
        in_specs=[
            pl.BlockSpec((blk, din), lambda i: (i, 0)),
            pl.BlockSpec((din, dmid), lambda i: (0, 0)),
            _vec_spec(dmid), _vec_spec(dmid), _vec_spec(dmid),
            pl.BlockSpec((dmid, dout), lambda i: (0, 0)),
            _vec_spec(dout),
        ],
        out_specs=pl.BlockSpec((2, dout), lambda i: (0, 0)),
        out_shape=jax.ShapeDtypeStruct((2, dout), jnp.float32),
        scratch_shapes=[pltpu.VMEM((2, dout), jnp.float32)],
    )(x, w1_t, b1, s1, t1, w2_t, b2)


def _run_echem(x, w1_t, b1, s1, t1, w2_t, b2, s2, t2):
    n, din = x.shape
    dmid = w1_t.shape[1]
    return pl.pallas_call(
        _echem_body,
        grid=(n // E_BLK,),
        in_specs=[
            pl.BlockSpec((E_BLK, din), lambda i: (i, 0)),
            pl.BlockSpec((din, dmid), lambda i: (0, 0)),
            _vec_spec(dmid), _vec_spec(dmid), _vec_spec(dmid),
            pl.BlockSpec((dmid, 2 * H), lambda i: (0, 0)),
            _vec_spec(2 * H), _vec_spec(2 * H), _vec_spec(2 * H),
        ],
        out_specs=pl.BlockSpec((E_BLK, H), lambda i: (i, 0)),
        out_shape=jax.ShapeDtypeStruct((n, H), jnp.float32),
    )(x, w1_t, b1, s1, t1, w2_t, b2, s2, t2)


def _copy_rows(src, s0, dst, d0, nrows):
    done = 0
    while done < nrows:
        n = min(128, nrows - done)
        pltpu.sync_copy(src.at[pl.ds(s0 + done, n)], dst.at[pl.ds(d0 + done, n)])
        done += n


def _sc_scatter_body(h_hbm, ids_hbm, bounds_hbm, out_hbm,
                     bounds_v, ids_v, idx_v, rows_v, acc,
                     sem_i0, sem_i1, sem_r0, sem_r1):
    core = lax.axis_index("c")
    sub = lax.axis_index("s")
    lane = lax.broadcasted_iota(jnp.int32, (16,), 0)
    sem_i = (sem_i0, sem_i1)
    sem_r = (sem_r0, sem_r1)

    pltpu.sync_copy(bounds_hbm, bounds_v)
    bvec = bounds_v[...]

    def run_part(part):
        base = PART_BASE[part]
        size = PART_BASE[part + 1] - base
        e_lo = bvec[part]
        e_hi = bvec[part + 1]

        def zero_row(r, carry):
            for j in range(H // 16):
                rows_v[0, r, pl.ds(j * 16, 16)] = jnp.zeros((16,), jnp.float32)
            return carry

        lax.fori_loop(0, T_EDGE, zero_row, 0)
        z0 = sub * ZCHUNK
        for q in range(ZCHUNK // T_EDGE):
            pltpu.sync_copy(rows_v.at[0],
                            acc.at[pl.ds(z0 + q * T_EDGE, T_EDGE)])
        plsc.subcore_barrier()

        e_lo_a = (e_lo // 8) * 8
        c8 = ((e_hi - e_lo_a + 16 * T_EDGE - 1) // (16 * T_EDGE)) * T_EDGE
        s_start = e_lo_a + sub * c8
        s_end = s_start + c8
        n_tiles = (c8 + T_EDGE - 1) // T_EDGE

        def tile_e(it):
            t = s_start + it * T_EDGE
            return jnp.minimum(t, N_EDGES - T_EDGE)

        def issue(it, b):
            e = tile_e(it)
            pltpu.async_copy(ids_hbm.at[pl.ds(e, T_EDGE)], ids_v.at[b], sem_i[b])
            pltpu.async_copy(h_hbm.at[pl.ds(e, T_EDGE)], rows_v.at[b], sem_r[b])

        def wait(b):
            pltpu.make_async_copy(ids_hbm.at[pl.ds(0, T_EDGE)], ids_v.at[b],
                                  sem_i[b]).wait()
            pltpu.make_async_copy(h_hbm.at[pl.ds(0, T_EDGE)], rows_v.at[b],
                                  sem_r[b]).wait()

        def process(it, b):
            t = s_start + it * T_EDGE
            e = tile_e(it)
            hi = jnp.minimum(t + T_EDGE, s_end)
            for j in range(T_EDGE // 16):
                pos = e + j * 16 + lane
                idv = ids_v[b, pl.ds(j * 16, 16)]
                valid = ((pos >= t) & (pos < hi) & (pos >= e_lo) & (pos < e_hi))
                idx_v[pl.ds(j * 16, 16)] = jnp.where(valid, idv - base, DUMMY_ROW)
            pltpu.sync_copy(rows_v.at[b], acc.at[idx_v], add=True)

        issue(0, 0)

        def pair_body(p, carry):
            i0 = 2 * p
            issue(jnp.minimum(i0 + 1, n_tiles - 1), 1)
            wait(0)
            process(i0, 0)
            issue(jnp.minimum(i0 + 2, n_tiles - 1), 0)
            wait(1)
            process(i0 + 1, 1)
            return carry

        lax.fori_loop(0, (n_tiles + 1) // 2, pair_body, 0)
        wait(0)
        plsc.subcore_barrier()

        wmain = 784
        wlast = size - 15 * wmain

        @pl.when(sub < 15)
        def _():
            _copy_rows(acc, sub * wmain, out_hbm, base + sub * wmain, wmain)

        @pl.when(sub == 15)
        def _():
            _copy_rows(acc, 15 * wmain, out_hbm, base + 15 * wmain, wlast)

        plsc.subcore_barrier()

    for part in range(SC_PARTS):
        @pl.when(core == part // 2)
        def _(part=part):
            run_part(part)


def _sc_segment_sum(h_chem, nbr_vids, bounds16):
    fn = pl.kernel(
        _sc_scatter_body,
        out_type=jax.ShapeDtypeStruct((N_NODES, H), jnp.float32),
        mesh=plsc.VectorSubcoreMesh(core_axis_name="c", subcore_axis_name="s",
                                    num_cores=2, num_subcores=16),
        scratch_types=[
            pltpu.VMEM((16,), jnp.int32),
            pltpu.VMEM((2, T_EDGE), jnp.int32),
            pltpu.VMEM((T_EDGE,), jnp.int32),
            pltpu.VMEM((2, T_EDGE, H), jnp.float32),
            pltpu.VMEM_SHARED((ACC_ROWS, H), jnp.float32),
            pltpu.SemaphoreType.DMA,
            pltpu.SemaphoreType.DMA,
            pltpu.SemaphoreType.DMA,
            pltpu.SemaphoreType.DMA,
        ],
    )
    return fn(h_chem, nbr_vids, bounds16)


def kernel(chem_feats, geom_feats, nbr_vids, cW1, cb1, cg1, cbeta1, cW2, cb2,
           cg2, cbeta2, gW1, gb1, gg1, gbeta1, gW2, gb2, gg2, gbeta2, fW1,
           fb1, fg1, fbeta1, fW2, fb2, fg2, fbeta2):
    def r(v):
        return v.reshape(1, -1)

    h_chem = _run_edge_mega(chem_feats, cW1.T, r(cb1), r(cg1), r(cbeta1),
                            cW2.T, r(cb2), r(cg2), r(cbeta2))

    parts = jnp.array(PART_BASE, dtype=jnp.int32)
    bounds = jnp.searchsorted(nbr_vids, parts).astype(jnp.int32)
    bounds16 = jnp.zeros((16,), jnp.int32).at[: SC_PARTS + 1].set(bounds)
    agg = _sc_segment_sum(h_chem, nbr_vids, bounds16)

    return _run_node_mega(geom_feats, agg,
                          gW1.T, r(gb1), r(gg1), r(gbeta1),
                          gW2.T, r(gb2), r(gg2), r(gbeta2),
                          fW1[:, :H].T, fW1[:, H:].T, r(fb1), r(fg1), r(fbeta1),
                          fW2.T, r(fb2), r(fg2), r(fbeta2))


# speedup vs baseline: 1.1173x; 1.1173x over previous
import jax
import jax.numpy as jnp
from jax import lax
from jax.experimental import pallas as pl
from jax.experimental.pallas import tpu as pltpu
from jax.experimental.pallas import tpu_sc as plsc

H = 128
NGDF = 16
NSIG = 16
CHEM_DIM = 2 + NGDF * 2
GEOM_DIM = NGDF * 2 + NSIG
N_NODES = 50000
N_EDGES = 800000
EPS = 1e-5

E_BLK = 8000
N_BLK = 5000

SC_PARTS = 4
PART_BASE = (0, 12512, 25024, 37536, N_NODES)
ACC_ROWS = 12544
DUMMY_ROW = 12512
ZCHUNK = 784
T_EDGE = 112


def _sigmoid(x):
    return 1.0 / (1.0 + jnp.exp(-x))


def _silu(x):
    return x * _sigmoid(x)


def _softplus(x):
    return jnp.maximum(x, 0.0) + jnp.log(1.0 + jnp.exp(-jnp.abs(x)))


def _dot(a, b):
    return jnp.dot(a, b, preferred_element_type=jnp.float32)


def _dot_hi(a, b):
    return jnp.dot(a, b, preferred_element_type=jnp.float32,
                   precision=lax.Precision.HIGHEST)


def _acc_stats(acc_ref, h):
    acc_ref[0:1, :] += jnp.sum(h, axis=0, keepdims=True)
    acc_ref[1:2, :] += jnp.sum(h * h, axis=0, keepdims=True)


def _mk_ss(ss_ref, acc_ref, g_ref, be_ref, n):
    mean = acc_ref[0:1, :] / n
    var = acc_ref[1:2, :] / n - mean * mean
    s = g_ref[...] / jnp.sqrt(var + EPS)
    ss_ref[0:1, :] = s
    ss_ref[1:2, :] = be_ref[...] - mean * s


def _edge_mega_body(x_ref, w1_ref, b1_ref, g1_ref, be1_ref,
                    w2_ref, b2_ref, g2_ref, be2_ref, o_ref,
                    acc1, acc2, ss1, ss2):
    p = pl.program_id(0)
    j = pl.program_id(1)

    @pl.when((p == 0) & (j == 0))
    def _():
        acc1[...] = jnp.zeros_like(acc1)

    @pl.when((p == 1) & (j == 0))
    def _():
        _mk_ss(ss1, acc1, g1_ref, be1_ref, N_EDGES)
        acc2[...] = jnp.zeros_like(acc2)

    @pl.when((p == 2) & (j == 0))
    def _():
        _mk_ss(ss2, acc2, g2_ref, be2_ref, N_EDGES)

    @pl.when(p == 0)
    def _():
        h = _dot(x_ref[...], w1_ref[...]) + b1_ref[...]
        _acc_stats(acc1, h)

    @pl.when(p == 1)
    def _():
        h1 = _dot(x_ref[...], w1_ref[...]) + b1_ref[...]
        a1 = _silu(h1 * ss1[0:1, :] + ss1[1:2, :])
        h2 = _dot(a1, w2_ref[...]) + b2_ref[...]
        _acc_stats(acc2, h2)

    @pl.when(p == 2)
    def _():
        h1 = _dot(x_ref[...], w1_ref[...]) + b1_ref[...]
        a1 = _silu(h1 * ss1[0:1, :] + ss1[1:2, :])
        h2 = (_dot(a1, w2_ref[...]) + b2_ref[...]) * ss2[0:1, :] + ss2[1:2, :]
        o_ref[...] = _sigmoid(h2[:, :H]) * _softplus(h2[:, H:])


def _run_edge_mega(x, w1t, b1, g1, be1, w2t, b2, g2, be2):
    n = x.shape[0]
    nb = n // E_BLK
    return pl.pallas_call(
        _edge_mega_body,
        grid=(3, nb),
        in_specs=[
            pl.BlockSpec((E_BLK, CHEM_DIM), lambda p, j: (j, 0)),
            pl.BlockSpec((CHEM_DIM, H), lambda p, j: (0, 0)),
            _evec(H), _evec(H), _evec(H),
            pl.BlockSpec((H, 2 * H), lambda p, j: (0, 0)),
            _evec(2 * H), _evec(2 * H), _evec(2 * H),
        ],
        out_specs=pl.BlockSpec((E_BLK, H),
                               lambda p, j: (jnp.where(p == 2, j, 0), 0)),
        out_shape=jax.ShapeDtypeStruct((n, H), jnp.float32),
        scratch_shapes=[
            pltpu.VMEM((2, H), jnp.float32),
            pltpu.VMEM((2, 2 * H), jnp.float32),
            pltpu.VMEM((2, H), jnp.float32),
            pltpu.VMEM((2, 2 * H), jnp.float32),
        ],
    )(x, w1t, b1, g1, be1, w2t, b2, g2, be2)


def _evec(d):
    return pl.BlockSpec((1, d), lambda p, j: (0, 0))


def _node_mega_body(g_ref, agg_ref, gw1_ref, gb1_ref, gg1_ref, gbe1_ref,
                    gw2_ref, gb2_ref, gg2_ref, gbe2_ref,
                    fwa_ref, fwb_ref, fb1_ref, fg1_ref, fbe1_ref,
                    fw2_ref, fb2_ref, fg2_ref, fbe2_ref, o_ref,
                    ag1, ag2, af1, af2, sg1, sg2, sf1, sf2):
    p = pl.program_id(0)
    j = pl.program_id(1)

    @pl.when((p == 0) & (j == 0))
    def _():
        ag1[...] = jnp.zeros_like(ag1)

    @pl.when((p == 1) & (j == 0))
    def _():
        _mk_ss(sg1, ag1, gg1_ref, gbe1_ref, N_NODES)
        ag2[...] = jnp.zeros_like(ag2)

    @pl.when((p == 2) & (j == 0))
    def _():
        _mk_ss(sg2, ag2, gg2_ref, gbe2_ref, N_NODES)
        af1[...] = jnp.zeros_like(af1)

    @pl.when((p == 3) & (j == 0))
    def _():
        _mk_ss(sf1, af1, fg1_ref, fbe1_ref, N_NODES)
        af2[...] = jnp.zeros_like(af2)

    @pl.when((p == 4) & (j == 0))
    def _():
        _mk_ss(sf2, af2, fg2_ref, fbe2_ref, N_NODES)

    def hg1_raw():
        return _dot(g_ref[...], gw1_ref[...]) + gb1_ref[...]

    def hg2n():
        a = _silu(hg1_raw() * sg1[0:1, :] + sg1[1:2, :])
        h = _dot(a, gw2_ref[...]) + gb2_ref[...]
        return h * sg2[0:1, :] + sg2[1:2, :]

    def f1_raw():
        return (_dot(agg_ref[...], fwa_ref[...])
                + _dot(hg2n(), fwb_ref[...]) + fb1_ref[...])

    def f2_raw():
        a = _silu(f1_raw() * sf1[0:1, :] + sf1[1:2, :])
        return _dot(a, fw2_ref[...]) + fb2_ref[...]

    @pl.when(p == 0)
    def _():
        _acc_stats(ag1, hg1_raw())

    @pl.when(p == 1)
    def _():
        a = _silu(hg1_raw() * sg1[0:1, :] + sg1[1:2, :])
        _acc_stats(ag2, _dot(a, gw2_ref[...]) + gb2_ref[...])

    @pl.when(p == 2)
    def _():
        _acc_stats(af1, f1_raw())

    @pl.when(p == 3)
    def _():
        _acc_stats(af2, f2_raw())

    @pl.when(p == 4)
    def _():
        o_ref[...] = f2_raw() * sf2[0:1, :] + sf2[1:2, :]


def _run_node_mega(g, agg, gw1t, gb1, gg1, gbe1, gw2t, gb2, gg2, gbe2,
                   fwa, fwb, fb1, fg1, fbe1, fw2t, fb2, fg2, fbe2):
    nb = N_NODES // N_BLK
    h2 = H // 2
    return pl.pallas_call(
        _node_mega_body,
        grid=(5, nb),
        in_specs=[
            pl.BlockSpec((N_BLK, GEOM_DIM), lambda p, j: (j, 0)),
            pl.BlockSpec((N_BLK, H), lambda p, j: (j, 0)),
            pl.BlockSpec((GEOM_DIM, h2), lambda p, j: (0, 0)),
            _evec(h2), _evec(h2), _evec(h2),
            pl.BlockSpec((h2, h2), lambda p, j: (0, 0)),
            _evec(h2), _evec(h2), _evec(h2),
            pl.BlockSpec((H, H), lambda p, j: (0, 0)),
            pl.BlockSpec((h2, H), lambda p, j: (0, 0)),
            _evec(H), _evec(H), _evec(H),
            pl.BlockSpec((H, H), lambda p, j: (0, 0)),
            _evec(H), _evec(H), _evec(H),
        ],
        out_specs=pl.BlockSpec((N_BLK, H),
                               lambda p, j: (jnp.where(p == 4, j, 0), 0)),
        out_shape=jax.ShapeDtypeStruct((N_NODES, H), jnp.float32),
        scratch_shapes=[
            pltpu.VMEM((2, h2), jnp.float32),
            pltpu.VMEM((2, h2), jnp.float32),
            pltpu.VMEM((2, H), jnp.float32),
            pltpu.VMEM((2, H), jnp.float32),
            pltpu.VMEM((2, h2), jnp.float32),
            pltpu.VMEM((2, h2), jnp.float32),
            pltpu.VMEM((2, H), jnp.float32),
            pltpu.VMEM((2, H), jnp.float32),
        ],
    )(g, agg, gw1t, gb1, gg1, gbe1, gw2t, gb2, gg2, gbe2,
      fwa, fwb, fb1, fg1, fbe1, fw2t, fb2, fg2, fbe2)


def _stats_mm1_body(x_ref, w_ref, b_ref, o_ref, acc_ref):
    i = pl.program_id(0)

    @pl.when(i == 0)
    def _():
        acc_ref[...] = jnp.zeros_like(acc_ref)

    h = _dot(x_ref[...], w_ref[...]) + b_ref[...]
    _acc_stats(acc_ref, h)

    @pl.when(i == pl.num_programs(0) - 1)
    def _():
        o_ref[...] = acc_ref[...]


def _stats_mm2_body(x_ref, w1_ref, b1_ref, s1_ref, t1_ref, w2_ref, b2_ref,
                    o_ref, acc_ref):
    i = pl.program_id(0)

    @pl.when(i == 0)
    def _():
        acc_ref[...] = jnp.zeros_like(acc_ref)

    h1 = _dot(x_ref[...], w1_ref[...]) + b1_ref[...]
    a1 = _silu(h1 * s1_ref[...] + t1_ref[...])
    h2 = _dot(a1, w2_ref[...]) + b2_ref[...]
    _acc_stats(acc_ref, h2)

    @pl.when(i == pl.num_programs(0) - 1)
    def _():
        o_ref[...] = acc_ref[...]


def _echem_body(x_ref, w1_ref, b1_ref, s1_ref, t1_ref, w2_ref, b2_ref,
                s2_ref, t2_ref, o_ref):
    h1 = _dot(x_ref[...], w1_ref[...]) + b1_ref[...]
    a1 = _silu(h1 * s1_ref[...] + t1_ref[...])
    h2 = (_dot(a1, w2_ref[...]) + b2_ref[...]) * s2_ref[...] + t2_ref[...]
    o_ref[...] = _sigmoid(h2[:, :H]) * _softplus(h2[:, H:])


def _vec_spec(d):
    return pl.BlockSpec((1, d), lambda i: (0, 0))


def _run_stats_mm1(x, w_t, b, blk):
    n, din = x.shape
    dout = w_t.shape[1]
    return pl.pallas_call(
        _stats_mm1_body,
        grid=(n // blk,),
        in_specs=[
            pl.BlockSpec((blk, din), lambda i: (i, 0)),
            pl.BlockSpec((din, dout), lambda i: (0, 0)),
            _vec_spec(dout),
        ],
        out_specs=pl.BlockSpec((2, dout), lambda i: (0, 0)),
        out_shape=jax.ShapeDtypeStruct((2, dout), jnp.float32),
        scratch_shapes=[pltpu.VMEM((2, dout), jnp.float32)],
    )(x, w_t, b)


def _run_stats_mm2(x, w1_t, b1, s1, t1, w2_t, b2, blk):
    n, din = x.shape
    dmid = w1_t.shape[1]
    dout = w2_t.shape[1]
    return pl.pallas_call(
        _stats_mm2_body,
        grid=(n // blk,),
        in_specs=[
            pl.BlockSpec((blk, din), lambda i: (i, 0)),
            pl.BlockSpec((din, dmid), lambda i: (0, 0)),
            _vec_spec(dmid), _vec_spec(dmid), _vec_spec(dmid),
            pl.BlockSpec((dmid, dout), lambda i: (0, 0)),
            _vec_spec(dout),
        ],
        out_specs=pl.BlockSpec((2, dout), lambda i: (0, 0)),
        out_shape=jax.ShapeDtypeStruct((2, dout), jnp.float32),
        scratch_shapes=[pltpu.VMEM((2, dout), jnp.float32)],
    )(x, w1_t, b1, s1, t1, w2_t, b2)


def _run_echem(x, w1_t, b1, s1, t1, w2_t, b2, s2, t2):
    n, din = x.shape
    dmid = w1_t.shape[1]
    return pl.pallas_call(
        _echem_body,
        grid=(n // E_BLK,),
        in_specs=[
            pl.BlockSpec((E_BLK, din), lambda i: (i, 0)),
            pl.BlockSpec((din, dmid), lambda i: (0, 0)),
            _vec_spec(dmid), _vec_spec(dmid), _vec_spec(dmid),
            pl.BlockSpec((dmid, 2 * H), lambda i: (0, 0)),
            _vec_spec(2 * H), _vec_spec(2 * H), _vec_spec(2 * H),
        ],
        out_specs=pl.BlockSpec((E_BLK, H), lambda i: (i, 0)),
        out_shape=jax.ShapeDtypeStruct((n, H), jnp.float32),
    )(x, w1_t, b1, s1, t1, w2_t, b2, s2, t2)


def _copy_rows(src, s0, dst, d0, nrows):
    done = 0
    while done < nrows:
        n = min(128, nrows - done)
        pltpu.sync_copy(src.at[pl.ds(s0 + done, n)], dst.at[pl.ds(d0 + done, n)])
        done += n


def _sc_scatter_body(h_hbm, ids_hbm, bounds_hbm, out_hbm,
                     bounds_v, ids_v, idx_v, rows_v, acc,
                     sem_i0, sem_i1, sem_r0, sem_r1):
    core = lax.axis_index("c")
    sub = lax.axis_index("s")
    lane = lax.broadcasted_iota(jnp.int32, (16,), 0)
    sem_i = (sem_i0, sem_i1)
    sem_r = (sem_r0, sem_r1)

    pltpu.sync_copy(bounds_hbm, bounds_v)
    bvec = bounds_v[...]

    def run_part(part):
        base = PART_BASE[part]
        size = PART_BASE[part + 1] - base
        e_lo = bvec[part]
        e_hi = bvec[part + 1]

        def zero_row(r, carry):
            for j in range(H // 16):
                rows_v[0, r, pl.ds(j * 16, 16)] = jnp.zeros((16,), jnp.float32)
            return carry

        lax.fori_loop(0, T_EDGE, zero_row, 0)
        z0 = sub * ZCHUNK
        for q in range(ZCHUNK // T_EDGE):
            pltpu.sync_copy(rows_v.at[0],
                            acc.at[pl.ds(z0 + q * T_EDGE, T_EDGE)])
        plsc.subcore_barrier()

        e_lo_a = (e_lo // 8) * 8
        c8 = ((e_hi - e_lo_a + 16 * T_EDGE - 1) // (16 * T_EDGE)) * T_EDGE
        s_start = e_lo_a + sub * c8
        s_end = s_start + c8
        n_tiles = (c8 + T_EDGE - 1) // T_EDGE

        def tile_e(it):
            t = s_start + it * T_EDGE
            return jnp.minimum(t, N_EDGES - T_EDGE)

        def issue(it, b):
            e = tile_e(it)
            pltpu.async_copy(ids_hbm.at[pl.ds(e, T_EDGE)], ids_v.at[b], sem_i[b])
            pltpu.async_copy(h_hbm.at[pl.ds(e, T_EDGE)], rows_v.at[b], sem_r[b])

        def wait(b):
            pltpu.make_async_copy(ids_hbm.at[pl.ds(0, T_EDGE)], ids_v.at[b],
                                  sem_i[b]).wait()
            pltpu.make_async_copy(h_hbm.at[pl.ds(0, T_EDGE)], rows_v.at[b],
                                  sem_r[b]).wait()

        def process(it, b):
            t = s_start + it * T_EDGE
            e = tile_e(it)
            hi = jnp.minimum(t + T_EDGE, s_end)
            for j in range(T_EDGE // 16):
                pos = e + j * 16 + lane
                idv = ids_v[b, pl.ds(j * 16, 16)]
                valid = ((pos >= t) & (pos < hi) & (pos >= e_lo) & (pos < e_hi))
                idx_v[pl.ds(j * 16, 16)] = jnp.where(valid, idv - base, DUMMY_ROW)
            pltpu.sync_copy(rows_v.at[b], acc.at[idx_v], add=True)

        issue(0, 0)

        def pair_body(p, carry):
            i0 = 2 * p
            issue(jnp.minimum(i0 + 1, n_tiles - 1), 1)
            wait(0)
            process(i0, 0)
            issue(jnp.minimum(i0 + 2, n_tiles - 1), 0)
            wait(1)
            process(i0 + 1, 1)
            return carry

        lax.fori_loop(0, (n_tiles + 1) // 2, pair_body, 0)
        wait(0)
        plsc.subcore_barrier()

        wmain = 784
        wlast = size - 15 * wmain

        @pl.when(sub < 15)
        def _():
            _copy_rows(acc, sub * wmain, out_hbm, base + sub * wmain, wmain)

        @pl.when(sub == 15)
        def _():
            _copy_rows(acc, 15 * wmain, out_hbm, base + 15 * wmain, wlast)

        plsc.subcore_barrier()

    for part in range(SC_PARTS):
        @pl.when(core == part // 2)
        def _(part=part):
            run_part(part)


def _sc_segment_sum(h_chem, nbr_vids, bounds16):
    fn = pl.kernel(
        _sc_scatter_body,
        out_type=jax.ShapeDtypeStruct((N_NODES, H), jnp.float32),
        mesh=plsc.VectorSubcoreMesh(core_axis_name="c", subcore_axis_name="s",
                                    num_cores=2, num_subcores=16),
        scratch_types=[
            pltpu.VMEM((16,), jnp.int32),
            pltpu.VMEM((2, T_EDGE), jnp.int32),
            pltpu.VMEM((T_EDGE,), jnp.int32),
            pltpu.VMEM((2, T_EDGE, H), jnp.float32),
            pltpu.VMEM_SHARED((ACC_ROWS, H), jnp.float32),
            pltpu.SemaphoreType.DMA,
            pltpu.SemaphoreType.DMA,
            pltpu.SemaphoreType.DMA,
            pltpu.SemaphoreType.DMA,
        ],
    )
    return fn(h_chem, nbr_vids, bounds16)


def kernel(chem_feats, geom_feats, nbr_vids, cW1, cb1, cg1, cbeta1, cW2, cb2,
           cg2, cbeta2, gW1, gb1, gg1, gbeta1, gW2, gb2, gg2, gbeta2, fW1,
           fb1, fg1, fbeta1, fW2, fb2, fg2, fbeta2):
    def r(v):
        return v.reshape(1, -1)

    h_chem = _run_edge_mega(chem_feats, cW1.T, r(cb1), r(cg1), r(cbeta1),
                            cW2.T, r(cb2), r(cg2), r(cbeta2))

    parts = jnp.array(PART_BASE, dtype=jnp.int32)
    bounds = jnp.searchsorted(nbr_vids, parts).astype(jnp.int32)
    bounds16 = jnp.zeros((16,), jnp.int32).at[: SC_PARTS + 1].set(bounds)
    agg = _sc_segment_sum(h_chem, nbr_vids, bounds16)

    return _run_node_mega(geom_feats, agg,
                          gW1.T, r(gb1), r(gg1), r(gbeta1),
                          gW2.T, r(gb2), r(gg2), r(gbeta2),
                          fW1[:, :H].T, fW1[:, H:].T, r(fb1), r(fg1), r(fbeta1),
                          fW2.T, r(fb2), r(fg2), r(fbeta2))


# speedup vs baseline: 1.1554x; 1.0341x over previous
import jax
import jax.numpy as jnp
from jax import lax
from jax.experimental import pallas as pl
from jax.experimental.pallas import tpu as pltpu
from jax.experimental.pallas import tpu_sc as plsc

H = 128
NGDF = 16
NSIG = 16
CHEM_DIM = 2 + NGDF * 2
GEOM_DIM = NGDF * 2 + NSIG
N_NODES = 50000
N_EDGES = 800000
EPS = 1e-5

E_BLK = 16000
N_BLK = 5000

SC_PARTS = 4
PART_BASE = (0, 12512, 25024, 37536, N_NODES)
ACC_ROWS = 12544
DUMMY_ROW = 12512
ZCHUNK = 784
T_EDGE = 112


def _sigmoid(x):
    return 1.0 / (1.0 + jnp.exp(-x))


def _silu(x):
    return x * _sigmoid(x)


def _softplus(x):
    return jnp.maximum(x, 0.0) + jnp.log(1.0 + jnp.exp(-jnp.abs(x)))


def _dot(a, b):
    return jnp.dot(a, b, preferred_element_type=jnp.float32)


def _dot_hi(a, b):
    return jnp.dot(a, b, preferred_element_type=jnp.float32,
                   precision=lax.Precision.HIGHEST)


def _acc_stats(acc_ref, h):
    acc_ref[0:1, :] += jnp.sum(h, axis=0, keepdims=True)
    acc_ref[1:2, :] += jnp.sum(h * h, axis=0, keepdims=True)


def _mk_ss(ss_ref, acc_ref, g_ref, be_ref, n):
    mean = acc_ref[0:1, :] / n
    var = acc_ref[1:2, :] / n - mean * mean
    s = g_ref[...] / jnp.sqrt(var + EPS)
    ss_ref[0:1, :] = s
    ss_ref[1:2, :] = be_ref[...] - mean * s


def _edge_mega_body(x_ref, w1_ref, b1_ref, g1_ref, be1_ref,
                    w2_ref, b2_ref, g2_ref, be2_ref, o_ref,
                    acc1, acc2, ss1, ss2):
    p = pl.program_id(0)
    j = pl.program_id(1)

    @pl.when((p == 0) & (j == 0))
    def _():
        acc1[...] = jnp.zeros_like(acc1)

    @pl.when((p == 1) & (j == 0))
    def _():
        _mk_ss(ss1, acc1, g1_ref, be1_ref, N_EDGES)
        acc2[...] = jnp.zeros_like(acc2)

    @pl.when((p == 2) & (j == 0))
    def _():
        _mk_ss(ss2, acc2, g2_ref, be2_ref, N_EDGES)

    @pl.when(p == 0)
    def _():
        h = _dot(x_ref[...], w1_ref[...]) + b1_ref[...]
        _acc_stats(acc1, h)

    @pl.when(p == 1)
    def _():
        h1 = _dot(x_ref[...], w1_ref[...]) + b1_ref[...]
        a1 = _silu(h1 * ss1[0:1, :] + ss1[1:2, :])
        h2 = _dot(a1, w2_ref[...]) + b2_ref[...]
        _acc_stats(acc2, h2)

    @pl.when(p == 2)
    def _():
        h1 = _dot(x_ref[...], w1_ref[...]) + b1_ref[...]
        a1 = _silu(h1 * ss1[0:1, :] + ss1[1:2, :])
        h2 = (_dot(a1, w2_ref[...]) + b2_ref[...]) * ss2[0:1, :] + ss2[1:2, :]
        o_ref[...] = _sigmoid(h2[:, :H]) * _softplus(h2[:, H:])


def _run_edge_mega(x, w1t, b1, g1, be1, w2t, b2, g2, be2):
    n = x.shape[0]
    nb = n // E_BLK
    return pl.pallas_call(
        _edge_mega_body,
        grid=(3, nb),
        in_specs=[
            pl.BlockSpec((E_BLK, CHEM_DIM), lambda p, j: (j, 0)),
            pl.BlockSpec((CHEM_DIM, H), lambda p, j: (0, 0)),
            _evec(H), _evec(H), _evec(H),
            pl.BlockSpec((H, 2 * H), lambda p, j: (0, 0)),
            _evec(2 * H), _evec(2 * H), _evec(2 * H),
        ],
        out_specs=pl.BlockSpec((E_BLK, H),
                               lambda p, j: (jnp.where(p == 2, j, 0), 0)),
        out_shape=jax.ShapeDtypeStruct((n, H), jnp.float32),
        scratch_shapes=[
            pltpu.VMEM((2, H), jnp.float32),
            pltpu.VMEM((2, 2 * H), jnp.float32),
            pltpu.VMEM((2, H), jnp.float32),
            pltpu.VMEM((2, 2 * H), jnp.float32),
        ],
    )(x, w1t, b1, g1, be1, w2t, b2, g2, be2)


def _evec(d):
    return pl.BlockSpec((1, d), lambda p, j: (0, 0))


def _node_mega_body(g_ref, agg_ref, gw1_ref, gb1_ref, gg1_ref, gbe1_ref,
                    gw2_ref, gb2_ref, gg2_ref, gbe2_ref,
                    fwa_ref, fwb_ref, fb1_ref, fg1_ref, fbe1_ref,
                    fw2_ref, fb2_ref, fg2_ref, fbe2_ref, o_ref,
                    ag1, ag2, af1, af2, sg1, sg2, sf1, sf2):
    p = pl.program_id(0)
    j = pl.program_id(1)

    @pl.when((p == 0) & (j == 0))
    def _():
        ag1[...] = jnp.zeros_like(ag1)

    @pl.when((p == 1) & (j == 0))
    def _():
        _mk_ss(sg1, ag1, gg1_ref, gbe1_ref, N_NODES)
        ag2[...] = jnp.zeros_like(ag2)

    @pl.when((p == 2) & (j == 0))
    def _():
        _mk_ss(sg2, ag2, gg2_ref, gbe2_ref, N_NODES)
        af1[...] = jnp.zeros_like(af1)

    @pl.when((p == 3) & (j == 0))
    def _():
        _mk_ss(sf1, af1, fg1_ref, fbe1_ref, N_NODES)
        af2[...] = jnp.zeros_like(af2)

    @pl.when((p == 4) & (j == 0))
    def _():
        _mk_ss(sf2, af2, fg2_ref, fbe2_ref, N_NODES)

    def hg1_raw():
        return _dot(g_ref[...], gw1_ref[...]) + gb1_ref[...]

    def hg2n():
        a = _silu(hg1_raw() * sg1[0:1, :] + sg1[1:2, :])
        h = _dot(a, gw2_ref[...]) + gb2_ref[...]
        return h * sg2[0:1, :] + sg2[1:2, :]

    def f1_raw():
        return (_dot(agg_ref[...], fwa_ref[...])
                + _dot(hg2n(), fwb_ref[...]) + fb1_ref[...])

    def f2_raw():
        a = _silu(f1_raw() * sf1[0:1, :] + sf1[1:2, :])
        return _dot(a, fw2_ref[...]) + fb2_ref[...]

    @pl.when(p == 0)
    def _():
        _acc_stats(ag1, hg1_raw())

    @pl.when(p == 1)
    def _():
        a = _silu(hg1_raw() * sg1[0:1, :] + sg1[1:2, :])
        _acc_stats(ag2, _dot(a, gw2_ref[...]) + gb2_ref[...])

    @pl.when(p == 2)
    def _():
        _acc_stats(af1, f1_raw())

    @pl.when(p == 3)
    def _():
        _acc_stats(af2, f2_raw())

    @pl.when(p == 4)
    def _():
        o_ref[...] = f2_raw() * sf2[0:1, :] + sf2[1:2, :]


def _run_node_mega(g, agg, gw1t, gb1, gg1, gbe1, gw2t, gb2, gg2, gbe2,
                   fwa, fwb, fb1, fg1, fbe1, fw2t, fb2, fg2, fbe2):
    nb = N_NODES // N_BLK
    h2 = H // 2
    return pl.pallas_call(
        _node_mega_body,
        grid=(5, nb),
        in_specs=[
            pl.BlockSpec((N_BLK, GEOM_DIM), lambda p, j: (j, 0)),
            pl.BlockSpec((N_BLK, H), lambda p, j: (j, 0)),
            pl.BlockSpec((GEOM_DIM, h2), lambda p, j: (0, 0)),
            _evec(h2), _evec(h2), _evec(h2),
            pl.BlockSpec((h2, h2), lambda p, j: (0, 0)),
            _evec(h2), _evec(h2), _evec(h2),
            pl.BlockSpec((H, H), lambda p, j: (0, 0)),
            pl.BlockSpec((h2, H), lambda p, j: (0, 0)),
            _evec(H), _evec(H), _evec(H),
            pl.BlockSpec((H, H), lambda p, j: (0, 0)),
            _evec(H), _evec(H), _evec(H),
        ],
        out_specs=pl.BlockSpec((N_BLK, H),
                               lambda p, j: (jnp.where(p == 4, j, 0), 0)),
        out_shape=jax.ShapeDtypeStruct((N_NODES, H), jnp.float32),
        scratch_shapes=[
            pltpu.VMEM((2, h2), jnp.float32),
            pltpu.VMEM((2, h2), jnp.float32),
            pltpu.VMEM((2, H), jnp.float32),
            pltpu.VMEM((2, H), jnp.float32),
            pltpu.VMEM((2, h2), jnp.float32),
            pltpu.VMEM((2, h2), jnp.float32),
            pltpu.VMEM((2, H), jnp.float32),
            pltpu.VMEM((2, H), jnp.float32),
        ],
    )(g, agg, gw1t, gb1, gg1, gbe1, gw2t, gb2, gg2, gbe2,
      fwa, fwb, fb1, fg1, fbe1, fw2t, fb2, fg2, fbe2)


def _stats_mm1_body(x_ref, w_ref, b_ref, o_ref, acc_ref):
    i = pl.program_id(0)

    @pl.when(i == 0)
    def _():
        acc_ref[...] = jnp.zeros_like(acc_ref)

    h = _dot(x_ref[...], w_ref[...]) + b_ref[...]
    _acc_stats(acc_ref, h)

    @pl.when(i == pl.num_programs(0) - 1)
    def _():
        o_ref[...] = acc_ref[...]


def _stats_mm2_body(x_ref, w1_ref, b1_ref, s1_ref, t1_ref, w2_ref, b2_ref,
                    o_ref, acc_ref):
    i = pl.program_id(0)

    @pl.when(i == 0)
    def _():
        acc_ref[...] = jnp.zeros_like(acc_ref)

    h1 = _dot(x_ref[...], w1_ref[...]) + b1_ref[...]
    a1 = _silu(h1 * s1_ref[...] + t1_ref[...])
    h2 = _dot(a1, w2_ref[...]) + b2_ref[...]
    _acc_stats(acc_ref, h2)

    @pl.when(i == pl.num_programs(0) - 1)
    def _():
        o_ref[...] = acc_ref[...]


def _echem_body(x_ref, w1_ref, b1_ref, s1_ref, t1_ref, w2_ref, b2_ref,
                s2_ref, t2_ref, o_ref):
    h1 = _dot(x_ref[...], w1_ref[...]) + b1_ref[...]
    a1 = _silu(h1 * s1_ref[...] + t1_ref[...])
    h2 = (_dot(a1, w2_ref[...]) + b2_ref[...]) * s2_ref[...] + t2_ref[...]
    o_ref[...] = _sigmoid(h2[:, :H]) * _softplus(h2[:, H:])


def _vec_spec(d):
    return pl.BlockSpec((1, d), lambda i: (0, 0))


def _run_stats_mm1(x, w_t, b, blk):
    n, din = x.shape
    dout = w_t.shape[1]
    return pl.pallas_call(
        _stats_mm1_body,
        grid=(n // blk,),
        in_specs=[
            pl.BlockSpec((blk, din), lambda i: (i, 0)),
            pl.BlockSpec((din, dout), lambda i: (0, 0)),
            _vec_spec(dout),
        ],
        out_specs=pl.BlockSpec((2, dout), lambda i: (0, 0)),
        out_shape=jax.ShapeDtypeStruct((2, dout), jnp.float32),
        scratch_shapes=[pltpu.VMEM((2, dout), jnp.float32)],
    )(x, w_t, b)


def _run_stats_mm2(x, w1_t, b1, s1, t1, w2_t, b2, blk):
    n, din = x.shape
    dmid = w1_t.shape[1]
    dout = w2_t.shape[1]
    return pl.pallas_call(
        _stats_mm2_body,
        grid=(n // blk,),
        in_specs=[
            pl.BlockSpec((blk, din), lambda i: (i, 0)),
            pl.BlockSpec((din, dmid), lambda i: (0, 0)),
            _vec_spec(dmid), _vec_spec(dmid), _vec_spec(dmid),
            pl.BlockSpec((dmid, dout), lambda i: (0, 0)),
            _vec_spec(dout),
        ],
        out_specs=pl.BlockSpec((2, dout), lambda i: (0, 0)),
        out_shape=jax.ShapeDtypeStruct((2, dout), jnp.float32),
        scratch_shapes=[pltpu.VMEM((2, dout), jnp.float32)],
    )(x, w1_t, b1, s1, t1, w2_t, b2)


def _run_echem(x, w1_t, b1, s1, t1, w2_t, b2, s2, t2):
    n, din = x.shape
    dmid = w1_t.shape[1]
    return pl.pallas_call(
        _echem_body,
        grid=(n // E_BLK,),
        in_specs=[
            pl.BlockSpec((E_BLK, din), lambda i: (i, 0)),
            pl.BlockSpec((din, dmid), lambda i: (0, 0)),
            _vec_spec(dmid), _vec_spec(dmid), _vec_spec(dmid),
            pl.BlockSpec((dmid, 2 * H), lambda i: (0, 0)),
            _vec_spec(2 * H), _vec_spec(2 * H), _vec_spec(2 * H),
        ],
        out_specs=pl.BlockSpec((E_BLK, H), lambda i: (i, 0)),
        out_shape=jax.ShapeDtypeStruct((n, H), jnp.float32),
    )(x, w1_t, b1, s1, t1, w2_t, b2, s2, t2)


def _copy_rows(src, s0, dst, d0, nrows):
    done = 0
    while done < nrows:
        n = min(128, nrows - done)
        pltpu.sync_copy(src.at[pl.ds(s0 + done, n)], dst.at[pl.ds(d0 + done, n)])
        done += n


def _sc_scatter_body(h_hbm, ids_hbm, bounds_hbm, out_hbm,
                     bounds_v, ids_v, idx_v, rows_v, acc,
                     sem_i0, sem_i1, sem_r0, sem_r1):
    core = lax.axis_index("c")
    sub = lax.axis_index("s")
    lane = lax.broadcasted_iota(jnp.int32, (16,), 0)
    sem_i = (sem_i0, sem_i1)
    sem_r = (sem_r0, sem_r1)

    pltpu.sync_copy(bounds_hbm, bounds_v)
    bvec = bounds_v[...]

    def run_part(part):
        base = PART_BASE[part]
        size = PART_BASE[part + 1] - base
        e_lo = bvec[part]
        e_hi = bvec[part + 1]

        def zero_row(r, carry):
            for j in range(H // 16):
                rows_v[0, r, pl.ds(j * 16, 16)] = jnp.zeros((16,), jnp.float32)
            return carry

        lax.fori_loop(0, T_EDGE, zero_row, 0)
        z0 = sub * ZCHUNK
        for q in range(ZCHUNK // T_EDGE):
            pltpu.sync_copy(rows_v.at[0],
                            acc.at[pl.ds(z0 + q * T_EDGE, T_EDGE)])
        plsc.subcore_barrier()

        e_lo_a = (e_lo // 8) * 8
        c8 = ((e_hi - e_lo_a + 16 * T_EDGE - 1) // (16 * T_EDGE)) * T_EDGE
        s_start = e_lo_a + sub * c8
        s_end = s_start + c8
        n_tiles = (c8 + T_EDGE - 1) // T_EDGE

        def tile_e(it):
            t = s_start + it * T_EDGE
            return jnp.minimum(t, N_EDGES - T_EDGE)

        def issue(it, b):
            e = tile_e(it)
            pltpu.async_copy(ids_hbm.at[pl.ds(e, T_EDGE)], ids_v.at[b], sem_i[b])
            pltpu.async_copy(h_hbm.at[pl.ds(e, T_EDGE)], rows_v.at[b], sem_r[b])

        def wait(b):
            pltpu.make_async_copy(ids_hbm.at[pl.ds(0, T_EDGE)], ids_v.at[b],
                                  sem_i[b]).wait()
            pltpu.make_async_copy(h_hbm.at[pl.ds(0, T_EDGE)], rows_v.at[b],
                                  sem_r[b]).wait()

        def process(it, b):
            t = s_start + it * T_EDGE
            e = tile_e(it)
            hi = jnp.minimum(t + T_EDGE, s_end)
            for j in range(T_EDGE // 16):
                pos = e + j * 16 + lane
                idv = ids_v[b, pl.ds(j * 16, 16)]
                valid = ((pos >= t) & (pos < hi) & (pos >= e_lo) & (pos < e_hi))
                idx_v[pl.ds(j * 16, 16)] = jnp.where(valid, idv - base, DUMMY_ROW)
            pltpu.sync_copy(rows_v.at[b], acc.at[idx_v], add=True)

        issue(0, 0)

        def pair_body(p, carry):
            i0 = 2 * p
            issue(jnp.minimum(i0 + 1, n_tiles - 1), 1)
            wait(0)
            process(i0, 0)
            issue(jnp.minimum(i0 + 2, n_tiles - 1), 0)
            wait(1)
            process(i0 + 1, 1)
            return carry

        lax.fori_loop(0, (n_tiles + 1) // 2, pair_body, 0)
        wait(0)
        plsc.subcore_barrier()

        wmain = 784
        wlast = size - 15 * wmain

        @pl.when(sub < 15)
        def _():
            _copy_rows(acc, sub * wmain, out_hbm, base + sub * wmain, wmain)

        @pl.when(sub == 15)
        def _():
            _copy_rows(acc, 15 * wmain, out_hbm, base + 15 * wmain, wlast)

        plsc.subcore_barrier()

    for part in range(SC_PARTS):
        @pl.when(core == part // 2)
        def _(part=part):
            run_part(part)


def _sc_segment_sum(h_chem, nbr_vids, bounds16):
    fn = pl.kernel(
        _sc_scatter_body,
        out_type=jax.ShapeDtypeStruct((N_NODES, H), jnp.float32),
        mesh=plsc.VectorSubcoreMesh(core_axis_name="c", subcore_axis_name="s",
                                    num_cores=2, num_subcores=16),
        scratch_types=[
            pltpu.VMEM((16,), jnp.int32),
            pltpu.VMEM((2, T_EDGE), jnp.int32),
            pltpu.VMEM((T_EDGE,), jnp.int32),
            pltpu.VMEM((2, T_EDGE, H), jnp.float32),
            pltpu.VMEM_SHARED((ACC_ROWS, H), jnp.float32),
            pltpu.SemaphoreType.DMA,
            pltpu.SemaphoreType.DMA,
            pltpu.SemaphoreType.DMA,
            pltpu.SemaphoreType.DMA,
        ],
    )
    return fn(h_chem, nbr_vids, bounds16)


def kernel(chem_feats, geom_feats, nbr_vids, cW1, cb1, cg1, cbeta1, cW2, cb2,
           cg2, cbeta2, gW1, gb1, gg1, gbeta1, gW2, gb2, gg2, gbeta2, fW1,
           fb1, fg1, fbeta1, fW2, fb2, fg2, fbeta2):
    def r(v):
        return v.reshape(1, -1)

    h_chem = _run_edge_mega(chem_feats, cW1.T, r(cb1), r(cg1), r(cbeta1),
                            cW2.T, r(cb2), r(cg2), r(cbeta2))

    parts = jnp.array(PART_BASE, dtype=jnp.int32)
    bounds = jnp.searchsorted(nbr_vids, parts).astype(jnp.int32)
    bounds16 = jnp.zeros((16,), jnp.int32).at[: SC_PARTS + 1].set(bounds)
    agg = _sc_segment_sum(h_chem, nbr_vids, bounds16)

    return _run_node_mega(geom_feats, agg,
                          gW1.T, r(gb1), r(gg1), r(gbeta1),
                          gW2.T, r(gb2), r(gg2), r(gbeta2),
                          fW1[:, :H].T, fW1[:, H:].T, r(fb1), r(fg1), r(fbeta1),
                          fW2.T, r(fb2), r(fg2), r(fbeta2))


# speedup vs baseline: 1.1626x; 1.0062x over previous
import jax
import jax.numpy as jnp
from jax import lax
from jax.experimental import pallas as pl
from jax.experimental.pallas import tpu as pltpu
from jax.experimental.pallas import tpu_sc as plsc

H = 128
NGDF = 16
NSIG = 16
CHEM_DIM = 2 + NGDF * 2
GEOM_DIM = NGDF * 2 + NSIG
N_NODES = 50000
N_EDGES = 800000
EPS = 1e-5

E_BLK = 20000
N_BLK = 5000

SC_PARTS = 4
PART_BASE = (0, 12512, 25024, 37536, N_NODES)
ACC_ROWS = 12544
DUMMY_ROW = 12512
ZCHUNK = 784
T_EDGE = 112


def _sigmoid(x):
    return 1.0 / (1.0 + jnp.exp(-x))


def _silu(x):
    return x * _sigmoid(x)


def _softplus(x):
    return jnp.maximum(x, 0.0) + jnp.log(1.0 + jnp.exp(-jnp.abs(x)))


def _dot(a, b):
    return jnp.dot(a, b, preferred_element_type=jnp.float32)


def _dot_hi(a, b):
    return jnp.dot(a, b, preferred_element_type=jnp.float32,
                   precision=lax.Precision.HIGHEST)


def _acc_stats(acc_ref, h):
    acc_ref[0:1, :] += jnp.sum(h, axis=0, keepdims=True)
    acc_ref[1:2, :] += jnp.sum(h * h, axis=0, keepdims=True)


def _mk_ss(ss_ref, acc_ref, g_ref, be_ref, n):
    mean = acc_ref[0:1, :] / n
    var = acc_ref[1:2, :] / n - mean * mean
    s = g_ref[...] / jnp.sqrt(var + EPS)
    ss_ref[0:1, :] = s
    ss_ref[1:2, :] = be_ref[...] - mean * s


def _edge_mega_body(x_ref, w1_ref, b1_ref, g1_ref, be1_ref,
                    w2_ref, b2_ref, g2_ref, be2_ref, o_ref,
                    acc1, acc2, ss1, ss2):
    p = pl.program_id(0)
    j = pl.program_id(1)

    @pl.when((p == 0) & (j == 0))
    def _():
        acc1[...] = jnp.zeros_like(acc1)

    @pl.when((p == 1) & (j == 0))
    def _():
        _mk_ss(ss1, acc1, g1_ref, be1_ref, N_EDGES)
        acc2[...] = jnp.zeros_like(acc2)

    @pl.when((p == 2) & (j == 0))
    def _():
        _mk_ss(ss2, acc2, g2_ref, be2_ref, N_EDGES)

    @pl.when(p == 0)
    def _():
        h = _dot(x_ref[...], w1_ref[...]) + b1_ref[...]
        _acc_stats(acc1, h)

    @pl.when(p == 1)
    def _():
        h1 = _dot(x_ref[...], w1_ref[...]) + b1_ref[...]
        a1 = _silu(h1 * ss1[0:1, :] + ss1[1:2, :])
        h2 = _dot(a1, w2_ref[...]) + b2_ref[...]
        _acc_stats(acc2, h2)

    @pl.when(p == 2)
    def _():
        h1 = _dot(x_ref[...], w1_ref[...]) + b1_ref[...]
        a1 = _silu(h1 * ss1[0:1, :] + ss1[1:2, :])
        h2 = (_dot(a1, w2_ref[...]) + b2_ref[...]) * ss2[0:1, :] + ss2[1:2, :]
        o_ref[...] = _sigmoid(h2[:, :H]) * _softplus(h2[:, H:])


def _run_edge_mega(x, w1t, b1, g1, be1, w2t, b2, g2, be2):
    n = x.shape[0]
    nb = n // E_BLK
    return pl.pallas_call(
        _edge_mega_body,
        grid=(3, nb),
        in_specs=[
            pl.BlockSpec((E_BLK, CHEM_DIM), lambda p, j: (j, 0)),
            pl.BlockSpec((CHEM_DIM, H), lambda p, j: (0, 0)),
            _evec(H), _evec(H), _evec(H),
            pl.BlockSpec((H, 2 * H), lambda p, j: (0, 0)),
            _evec(2 * H), _evec(2 * H), _evec(2 * H),
        ],
        out_specs=pl.BlockSpec((E_BLK, H),
                               lambda p, j: (jnp.where(p == 2, j, 0), 0)),
        out_shape=jax.ShapeDtypeStruct((n, H), jnp.float32),
        scratch_shapes=[
            pltpu.VMEM((2, H), jnp.float32),
            pltpu.VMEM((2, 2 * H), jnp.float32),
            pltpu.VMEM((2, H), jnp.float32),
            pltpu.VMEM((2, 2 * H), jnp.float32),
        ],
    )(x, w1t, b1, g1, be1, w2t, b2, g2, be2)


def _evec(d):
    return pl.BlockSpec((1, d), lambda p, j: (0, 0))


def _node_mega_body(g_ref, agg_ref, gw1_ref, gb1_ref, gg1_ref, gbe1_ref,
                    gw2_ref, gb2_ref, gg2_ref, gbe2_ref,
                    fwa_ref, fwb_ref, fb1_ref, fg1_ref, fbe1_ref,
                    fw2_ref, fb2_ref, fg2_ref, fbe2_ref, o_ref,
                    ag1, ag2, af1, af2, sg1, sg2, sf1, sf2):
    p = pl.program_id(0)
    j = pl.program_id(1)

    @pl.when((p == 0) & (j == 0))
    def _():
        ag1[...] = jnp.zeros_like(ag1)

    @pl.when((p == 1) & (j == 0))
    def _():
        _mk_ss(sg1, ag1, gg1_ref, gbe1_ref, N_NODES)
        ag2[...] = jnp.zeros_like(ag2)

    @pl.when((p == 2) & (j == 0))
    def _():
        _mk_ss(sg2, ag2, gg2_ref, gbe2_ref, N_NODES)
        af1[...] = jnp.zeros_like(af1)

    @pl.when((p == 3) & (j == 0))
    def _():
        _mk_ss(sf1, af1, fg1_ref, fbe1_ref, N_NODES)
        af2[...] = jnp.zeros_like(af2)

    @pl.when((p == 4) & (j == 0))
    def _():
        _mk_ss(sf2, af2, fg2_ref, fbe2_ref, N_NODES)

    def hg1_raw():
        return _dot(g_ref[...], gw1_ref[...]) + gb1_ref[...]

    def hg2n():
        a = _silu(hg1_raw() * sg1[0:1, :] + sg1[1:2, :])
        h = _dot(a, gw2_ref[...]) + gb2_ref[...]
        return h * sg2[0:1, :] + sg2[1:2, :]

    def f1_raw():
        return (_dot(agg_ref[...], fwa_ref[...])
                + _dot(hg2n(), fwb_ref[...]) + fb1_ref[...])

    def f2_raw():
        a = _silu(f1_raw() * sf1[0:1, :] + sf1[1:2, :])
        return _dot(a, fw2_ref[...]) + fb2_ref[...]

    @pl.when(p == 0)
    def _():
        _acc_stats(ag1, hg1_raw())

    @pl.when(p == 1)
    def _():
        a = _silu(hg1_raw() * sg1[0:1, :] + sg1[1:2, :])
        _acc_stats(ag2, _dot(a, gw2_ref[...]) + gb2_ref[...])

    @pl.when(p == 2)
    def _():
        _acc_stats(af1, f1_raw())

    @pl.when(p == 3)
    def _():
        _acc_stats(af2, f2_raw())

    @pl.when(p == 4)
    def _():
        o_ref[...] = f2_raw() * sf2[0:1, :] + sf2[1:2, :]


def _run_node_mega(g, agg, gw1t, gb1, gg1, gbe1, gw2t, gb2, gg2, gbe2,
                   fwa, fwb, fb1, fg1, fbe1, fw2t, fb2, fg2, fbe2):
    nb = N_NODES // N_BLK
    h2 = H // 2
    return pl.pallas_call(
        _node_mega_body,
        grid=(5, nb),
        in_specs=[
            pl.BlockSpec((N_BLK, GEOM_DIM), lambda p, j: (j, 0)),
            pl.BlockSpec((N_BLK, H), lambda p, j: (j, 0)),
            pl.BlockSpec((GEOM_DIM, h2), lambda p, j: (0, 0)),
            _evec(h2), _evec(h2), _evec(h2),
            pl.BlockSpec((h2, h2), lambda p, j: (0, 0)),
            _evec(h2), _evec(h2), _evec(h2),
            pl.BlockSpec((H, H), lambda p, j: (0, 0)),
            pl.BlockSpec((h2, H), lambda p, j: (0, 0)),
            _evec(H), _evec(H), _evec(H),
            pl.BlockSpec((H, H), lambda p, j: (0, 0)),
            _evec(H), _evec(H), _evec(H),
        ],
        out_specs=pl.BlockSpec((N_BLK, H),
                               lambda p, j: (jnp.where(p == 4, j, 0), 0)),
        out_shape=jax.ShapeDtypeStruct((N_NODES, H), jnp.float32),
        scratch_shapes=[
            pltpu.VMEM((2, h2), jnp.float32),
            pltpu.VMEM((2, h2), jnp.float32),
            pltpu.VMEM((2, H), jnp.float32),
            pltpu.VMEM((2, H), jnp.float32),
            pltpu.VMEM((2, h2), jnp.float32),
            pltpu.VMEM((2, h2), jnp.float32),
            pltpu.VMEM((2, H), jnp.float32),
            pltpu.VMEM((2, H), jnp.float32),
        ],
    )(g, agg, gw1t, gb1, gg1, gbe1, gw2t, gb2, gg2, gbe2,
      fwa, fwb, fb1, fg1, fbe1, fw2t, fb2, fg2, fbe2)


def _stats_mm1_body(x_ref, w_ref, b_ref, o_ref, acc_ref):
    i = pl.program_id(0)

    @pl.when(i == 0)
    def _():
        acc_ref[...] = jnp.zeros_like(acc_ref)

    h = _dot(x_ref[...], w_ref[...]) + b_ref[...]
    _acc_stats(acc_ref, h)

    @pl.when(i == pl.num_programs(0) - 1)
    def _():
        o_ref[...] = acc_ref[...]


def _stats_mm2_body(x_ref, w1_ref, b1_ref, s1_ref, t1_ref, w2_ref, b2_ref,
                    o_ref, acc_ref):
    i = pl.program_id(0)

    @pl.when(i == 0)
    def _():
        acc_ref[...] = jnp.zeros_like(acc_ref)

    h1 = _dot(x_ref[...], w1_ref[...]) + b1_ref[...]
    a1 = _silu(h1 * s1_ref[...] + t1_ref[...])
    h2 = _dot(a1, w2_ref[...]) + b2_ref[...]
    _acc_stats(acc_ref, h2)

    @pl.when(i == pl.num_programs(0) - 1)
    def _():
        o_ref[...] = acc_ref[...]


def _echem_body(x_ref, w1_ref, b1_ref, s1_ref, t1_ref, w2_ref, b2_ref,
                s2_ref, t2_ref, o_ref):
    h1 = _dot(x_ref[...], w1_ref[...]) + b1_ref[...]
    a1 = _silu(h1 * s1_ref[...] + t1_ref[...])
    h2 = (_dot(a1, w2_ref[...]) + b2_ref[...]) * s2_ref[...] + t2_ref[...]
    o_ref[...] = _sigmoid(h2[:, :H]) * _softplus(h2[:, H:])


def _vec_spec(d):
    return pl.BlockSpec((1, d), lambda i: (0, 0))


def _run_stats_mm1(x, w_t, b, blk):
    n, din = x.shape
    dout = w_t.shape[1]
    return pl.pallas_call(
        _stats_mm1_body,
        grid=(n // blk,),
        in_specs=[
            pl.BlockSpec((blk, din), lambda i: (i, 0)),
            pl.BlockSpec((din, dout), lambda i: (0, 0)),
            _vec_spec(dout),
        ],
        out_specs=pl.BlockSpec((2, dout), lambda i: (0, 0)),
        out_shape=jax.ShapeDtypeStruct((2, dout), jnp.float32),
        scratch_shapes=[pltpu.VMEM((2, dout), jnp.float32)],
    )(x, w_t, b)


def _run_stats_mm2(x, w1_t, b1, s1, t1, w2_t, b2, blk):
    n, din = x.shape
    dmid = w1_t.shape[1]
    dout = w2_t.shape[1]
    return pl.pallas_call(
        _stats_mm2_body,
        grid=(n // blk,),
        in_specs=[
            pl.BlockSpec((blk, din), lambda i: (i, 0)),
            pl.BlockSpec((din, dmid), lambda i: (0, 0)),
            _vec_spec(dmid), _vec_spec(dmid), _vec_spec(dmid),
            pl.BlockSpec((dmid, dout), lambda i: (0, 0)),
            _vec_spec(dout),
        ],
        out_specs=pl.BlockSpec((2, dout), lambda i: (0, 0)),
        out_shape=jax.ShapeDtypeStruct((2, dout), jnp.float32),
        scratch_shapes=[pltpu.VMEM((2, dout), jnp.float32)],
    )(x, w1_t, b1, s1, t1, w2_t, b2)


def _run_echem(x, w1_t, b1, s1, t1, w2_t, b2, s2, t2):
    n, din = x.shape
    dmid = w1_t.shape[1]
    return pl.pallas_call(
        _echem_body,
        grid=(n // E_BLK,),
        in_specs=[
            pl.BlockSpec((E_BLK, din), lambda i: (i, 0)),
            pl.BlockSpec((din, dmid), lambda i: (0, 0)),
            _vec_spec(dmid), _vec_spec(dmid), _vec_spec(dmid),
            pl.BlockSpec((dmid, 2 * H), lambda i: (0, 0)),
            _vec_spec(2 * H), _vec_spec(2 * H), _vec_spec(2 * H),
        ],
        out_specs=pl.BlockSpec((E_BLK, H), lambda i: (i, 0)),
        out_shape=jax.ShapeDtypeStruct((n, H), jnp.float32),
    )(x, w1_t, b1, s1, t1, w2_t, b2, s2, t2)


def _copy_rows(src, s0, dst, d0, nrows):
    done = 0
    while done < nrows:
        n = min(128, nrows - done)
        pltpu.sync_copy(src.at[pl.ds(s0 + done, n)], dst.at[pl.ds(d0 + done, n)])
        done += n


def _sc_scatter_body(h_hbm, ids_hbm, bounds_hbm, out_hbm,
                     bounds_v, ids_v, idx_v, rows_v, acc,
                     sem_i0, sem_i1, sem_r0, sem_r1):
    core = lax.axis_index("c")
    sub = lax.axis_index("s")
    lane = lax.broadcasted_iota(jnp.int32, (16,), 0)
    sem_i = (sem_i0, sem_i1)
    sem_r = (sem_r0, sem_r1)

    pltpu.sync_copy(bounds_hbm, bounds_v)
    bvec = bounds_v[...]

    def run_part(part):
        base = PART_BASE[part]
        size = PART_BASE[part + 1] - base
        e_lo = bvec[part]
        e_hi = bvec[part + 1]

        def zero_row(r, carry):
            for j in range(H // 16):
                rows_v[0, r, pl.ds(j * 16, 16)] = jnp.zeros((16,), jnp.float32)
            return carry

        lax.fori_loop(0, T_EDGE, zero_row, 0)
        z0 = sub * ZCHUNK
        for q in range(ZCHUNK // T_EDGE):
            pltpu.sync_copy(rows_v.at[0],
                            acc.at[pl.ds(z0 + q * T_EDGE, T_EDGE)])
        plsc.subcore_barrier()

        e_lo_a = (e_lo // 8) * 8
        c8 = ((e_hi - e_lo_a + 16 * T_EDGE - 1) // (16 * T_EDGE)) * T_EDGE
        s_start = e_lo_a + sub * c8
        s_end = s_start + c8
        n_tiles = (c8 + T_EDGE - 1) // T_EDGE

        def tile_e(it):
            t = s_start + it * T_EDGE
            return jnp.minimum(t, N_EDGES - T_EDGE)

        def issue(it, b):
            e = tile_e(it)
            pltpu.async_copy(ids_hbm.at[pl.ds(e, T_EDGE)], ids_v.at[b], sem_i[b])
            pltpu.async_copy(h_hbm.at[pl.ds(e, T_EDGE)], rows_v.at[b], sem_r[b])

        def wait(b):
            pltpu.make_async_copy(ids_hbm.at[pl.ds(0, T_EDGE)], ids_v.at[b],
                                  sem_i[b]).wait()
            pltpu.make_async_copy(h_hbm.at[pl.ds(0, T_EDGE)], rows_v.at[b],
                                  sem_r[b]).wait()

        def process(it, b):
            t = s_start + it * T_EDGE
            e = tile_e(it)
            hi = jnp.minimum(t + T_EDGE, s_end)
            for j in range(T_EDGE // 16):
                pos = e + j * 16 + lane
                idv = ids_v[b, pl.ds(j * 16, 16)]
                valid = ((pos >= t) & (pos < hi) & (pos >= e_lo) & (pos < e_hi))
                idx_v[pl.ds(j * 16, 16)] = jnp.where(valid, idv - base, DUMMY_ROW)
            pltpu.sync_copy(rows_v.at[b], acc.at[idx_v], add=True)

        issue(0, 0)

        def pair_body(p, carry):
            i0 = 2 * p
            issue(jnp.minimum(i0 + 1, n_tiles - 1), 1)
            wait(0)
            process(i0, 0)
            issue(jnp.minimum(i0 + 2, n_tiles - 1), 0)
            wait(1)
            process(i0 + 1, 1)
            return carry

        lax.fori_loop(0, (n_tiles + 1) // 2, pair_body, 0)
        wait(0)
        plsc.subcore_barrier()

        wmain = 784
        wlast = size - 15 * wmain

        @pl.when(sub < 15)
        def _():
            _copy_rows(acc, sub * wmain, out_hbm, base + sub * wmain, wmain)

        @pl.when(sub == 15)
        def _():
            _copy_rows(acc, 15 * wmain, out_hbm, base + 15 * wmain, wlast)

        plsc.subcore_barrier()

    for part in range(SC_PARTS):
        @pl.when(core == part // 2)
        def _(part=part):
            run_part(part)


def _sc_segment_sum(h_chem, nbr_vids, bounds16):
    fn = pl.kernel(
        _sc_scatter_body,
        out_type=jax.ShapeDtypeStruct((N_NODES, H), jnp.float32),
        mesh=plsc.VectorSubcoreMesh(core_axis_name="c", subcore_axis_name="s",
                                    num_cores=2, num_subcores=16),
        scratch_types=[
            pltpu.VMEM((16,), jnp.int32),
            pltpu.VMEM((2, T_EDGE), jnp.int32),
            pltpu.VMEM((T_EDGE,), jnp.int32),
            pltpu.VMEM((2, T_EDGE, H), jnp.float32),
            pltpu.VMEM_SHARED((ACC_ROWS, H), jnp.float32),
            pltpu.SemaphoreType.DMA,
            pltpu.SemaphoreType.DMA,
            pltpu.SemaphoreType.DMA,
            pltpu.SemaphoreType.DMA,
        ],
    )
    return fn(h_chem, nbr_vids, bounds16)


def kernel(chem_feats, geom_feats, nbr_vids, cW1, cb1, cg1, cbeta1, cW2, cb2,
           cg2, cbeta2, gW1, gb1, gg1, gbeta1, gW2, gb2, gg2, gbeta2, fW1,
           fb1, fg1, fbeta1, fW2, fb2, fg2, fbeta2):
    def r(v):
        return v.reshape(1, -1)

    h_chem = _run_edge_mega(chem_feats, cW1.T, r(cb1), r(cg1), r(cbeta1),
                            cW2.T, r(cb2), r(cg2), r(cbeta2))

    parts = jnp.array(PART_BASE, dtype=jnp.int32)
    bounds = jnp.searchsorted(nbr_vids, parts).astype(jnp.int32)
    bounds16 = jnp.zeros((16,), jnp.int32).at[: SC_PARTS + 1].set(bounds)
    agg = _sc_segment_sum(h_chem, nbr_vids, bounds16)

    return _run_node_mega(geom_feats, agg,
                          gW1.T, r(gb1), r(gg1), r(gbeta1),
                          gW2.T, r(gb2), r(gg2), r(gbeta2),
                          fW1[:, :H].T, fW1[:, H:].T, r(fb1), r(fg1), r(fbeta1),
                          fW2.T, r(fb2), r(fg2), r(fbeta2))
